# baseline (speedup 1.0000x reference)
import jax
import jax.numpy as jnp
from jax.experimental import pallas as pl
from jax.experimental.pallas import tpu as pltpu

N, T, FEAT, C, U, V = 4, 512, 80, 256, 48, 256
BLANK = 0
UP = 56
TB = 64
NEG = -1e30
D_TOT = T + U + 1


def _phase1(x_ref, we_ref, be_ref, oh_ref, ohs_ref, emb_ref, wj_ref, bj_ref,
            lpb_ref, lps_ref, dec_ref):
    xb = x_ref[0]
    enc = jnp.dot(xb, we_ref[...], preferred_element_type=jnp.float32) + be_ref[0]

    @pl.when(pl.program_id(1) == 0)
    def _():
        oh = oh_ref[0]
        dec_ref[...] = jnp.dot(oh, emb_ref[...],
                               preferred_element_type=jnp.float32)

    dec = dec_ref[...]
    joint = jnp.tanh(enc[:, None, :] + dec[None, :, :])
    logits = jnp.dot(joint.reshape(TB * UP, C), wj_ref[...],
                     preferred_element_type=jnp.float32) + bj_ref[0]
    l3 = logits.reshape(TB, UP, V)
    m = jnp.max(l3, axis=2)
    lse = m + jnp.log(jnp.sum(jnp.exp(l3 - m[:, :, None]), axis=2))
    iota_v = jax.lax.broadcasted_iota(jnp.int32, (TB, UP, V), 2)
    lpb = jnp.sum(jnp.where(iota_v == BLANK, l3, 0.0), axis=2) - lse
    sym = jnp.sum(l3 * ohs_ref[0][None], axis=2) - lse
    ucol = jax.lax.broadcasted_iota(jnp.int32, (TB, UP), 1)
    lpb_ref[0] = jnp.where(ucol <= U, lpb, NEG)
    lps_ref[0] = jnp.where(ucol < U, sym, NEG)


def _phase2(pk_ref, out_ref):
    u_iota = jax.lax.broadcasted_iota(jnp.int32, (N, UP), 1)
    a0 = jnp.where(u_iota == 0, 0.0, NEG)
    out_ref[0] = a0

    def body(d, a):
        pk = pk_ref[d]
        sb = pk[:N]
        ss = pk[N:]
        a_sh = jnp.concatenate(
            [jnp.full((N, 1), NEG, jnp.float32), a[:, :-1]], axis=1)
        t1 = a + sb
        t2 = a_sh + ss
        mx = jnp.maximum(t1, t2)
        mn = jnp.minimum(t1, t2)
        a_new = mx + jnp.log1p(jnp.exp(mn - mx))
        out_ref[d] = a_new
        return a_new

    jax.lax.fori_loop(1, D_TOT, body, a0, unroll=8)


def _shear(arrT, left_pad, width):
    w = width + left_pad
    p = jnp.pad(arrT, ((0, 0), (0, 0), (left_pad, (w + UP + 1) - w)),
                constant_values=NEG)
    flat = p.reshape(N, UP * (w + UP + 1))[:, :UP * (w + UP)]
    sh = flat.reshape(N, UP, w + UP)[:, :, :D_TOT]
    return jnp.transpose(sh, (2, 0, 1))


def kernel(x, x_lens, y_padded, y_lens, W_enc, b_enc, emb, W_join, b_join):
    f32 = jnp.float32
    sos_y = jnp.concatenate(
        [jnp.zeros((N, 1), y_padded.dtype), y_padded], axis=1)
    sos_pad = jnp.pad(sos_y, ((0, 0), (0, UP - (U + 1))))
    vio = jnp.arange(V, dtype=sos_pad.dtype)
    oh = (sos_pad[:, :, None] == vio).astype(f32)
    yp_pad = jnp.pad(y_padded, ((0, 0), (0, UP - U)), constant_values=-1)
    ohs = (yp_pad[:, :, None] == vio).astype(f32)

    lpb, lps = pl.pallas_call(
        _phase1,
        grid=(N, T // TB),
        in_specs=[
            pl.BlockSpec((1, TB, FEAT), lambda n, t: (n, t, 0)),
            pl.BlockSpec((FEAT, C), lambda n, t: (0, 0)),
            pl.BlockSpec((1, C), lambda n, t: (0, 0)),
            pl.BlockSpec((1, UP, V), lambda n, t: (n, 0, 0)),
            pl.BlockSpec((1, UP, V), lambda n, t: (n, 0, 0)),
            pl.BlockSpec((V, C), lambda n, t: (0, 0)),
            pl.BlockSpec((C, V), lambda n, t: (0, 0)),
            pl.BlockSpec((1, V), lambda n, t: (0, 0)),
        ],
        out_specs=[
            pl.BlockSpec((1, TB, UP), lambda n, t: (n, t, 0)),
            pl.BlockSpec((1, TB, UP), lambda n, t: (n, t, 0)),
        ],
        out_shape=[
            jax.ShapeDtypeStruct((N, T, UP), f32),
            jax.ShapeDtypeStruct((N, T, UP), f32),
        ],
        scratch_shapes=[pltpu.VMEM((UP, C), jnp.float32)],
    )(x.astype(f32), W_enc.astype(f32), b_enc.reshape(1, C).astype(f32),
      oh, ohs, emb.astype(f32), W_join.astype(f32),
      b_join.reshape(1, V).astype(f32))

    sb = _shear(jnp.transpose(lpb, (0, 2, 1)), 1, T)
    lpsT = jnp.transpose(lps, (0, 2, 1))
    ls2 = jnp.pad(lpsT, ((0, 0), (1, 0), (0, 0)),
                  constant_values=NEG)[:, :UP]
    ss = _shear(ls2, 0, T)

    packed = jnp.concatenate([sb, ss], axis=1)

    alphas = pl.pallas_call(
        _phase2,
        out_shape=jax.ShapeDtypeStruct((D_TOT, N, UP), f32),
    )(packed)

    n_idx = jnp.arange(N)
    dn = x_lens - 1 + y_lens
    final_alpha = alphas[dn, n_idx, y_lens]
    final_blank = lpb[n_idx, x_lens - 1, y_lens]
    return -jnp.sum(final_alpha + final_blank)


# speedup vs baseline: 1.0212x; 1.0212x over previous
import jax
import jax.numpy as jnp
from jax.experimental import pallas as pl
from jax.experimental.pallas import tpu as pltpu

N, T, FEAT, C, U, V = 4, 512, 80, 256, 48, 256
BLANK = 0
UP = 56
TB = 128
NEG = -1e30
D_TOT = T + U + 1


def _phase1(x_ref, we_ref, be_ref, oh_ref, ohs_ref, emb_ref, wj_ref, bj_ref,
            lpb_ref, lps_ref, dec_ref):
    xb = x_ref[0]
    enc = jnp.dot(xb, we_ref[...], preferred_element_type=jnp.float32) + be_ref[0]

    @pl.when(pl.program_id(1) == 0)
    def _():
        oh = oh_ref[0]
        dec_ref[...] = jnp.dot(oh, emb_ref[...],
                               preferred_element_type=jnp.float32)

    dec = dec_ref[...]
    joint = jnp.tanh(enc.astype(jnp.bfloat16)[:, None, :]
                     + dec.astype(jnp.bfloat16)[None, :, :])
    logits = jnp.dot(joint.reshape(TB * UP, C), wj_ref[...],
                     preferred_element_type=jnp.float32) + bj_ref[0]
    l3 = logits.reshape(TB, UP, V)
    m = jnp.max(l3, axis=2)
    lse = m + jnp.log(jnp.sum(jnp.exp(l3 - m[:, :, None]), axis=2))
    iota_v = jax.lax.broadcasted_iota(jnp.int32, (TB, UP, V), 2)
    lpb = jnp.sum(jnp.where(iota_v == BLANK, l3, 0.0), axis=2) - lse
    sym = jnp.sum(l3 * ohs_ref[0][None], axis=2) - lse
    ucol = jax.lax.broadcasted_iota(jnp.int32, (TB, UP), 1)
    lpb_ref[0] = jnp.where(ucol <= U, lpb, NEG)
    lps_ref[0] = jnp.where(ucol < U, sym, NEG)


def _phase2(pk_ref, out_ref):
    u_iota = jax.lax.broadcasted_iota(jnp.int32, (N, UP), 1)
    a0 = jnp.where(u_iota == 0, 0.0, NEG)
    out_ref[0] = a0

    def body(d, a):
        pk = pk_ref[d]
        sb = pk[:N]
        ss = pk[N:]
        a_sh = jnp.concatenate(
            [jnp.full((N, 1), NEG, jnp.float32), a[:, :-1]], axis=1)
        t1 = a + sb
        t2 = a_sh + ss
        mx = jnp.maximum(t1, t2)
        mn = jnp.minimum(t1, t2)
        a_new = mx + jnp.log1p(jnp.exp(mn - mx))
        out_ref[d] = a_new
        return a_new

    jax.lax.fori_loop(1, D_TOT, body, a0, unroll=8)


def _shear(arrT, left_pad, width):
    w = width + left_pad
    p = jnp.pad(arrT, ((0, 0), (0, 0), (left_pad, (w + UP + 1) - w)),
                constant_values=NEG)
    flat = p.reshape(N, UP * (w + UP + 1))[:, :UP * (w + UP)]
    sh = flat.reshape(N, UP, w + UP)[:, :, :D_TOT]
    return jnp.transpose(sh, (2, 0, 1))


def kernel(x, x_lens, y_padded, y_lens, W_enc, b_enc, emb, W_join, b_join):
    f32 = jnp.float32
    sos_y = jnp.concatenate(
        [jnp.zeros((N, 1), y_padded.dtype), y_padded], axis=1)
    sos_pad = jnp.pad(sos_y, ((0, 0), (0, UP - (U + 1))))
    vio = jnp.arange(V, dtype=sos_pad.dtype)
    oh = (sos_pad[:, :, None] == vio).astype(f32)
    yp_pad = jnp.pad(y_padded, ((0, 0), (0, UP - U)), constant_values=-1)
    ohs = (yp_pad[:, :, None] == vio).astype(f32)

    lpb, lps = pl.pallas_call(
        _phase1,
        grid=(N, T // TB),
        in_specs=[
            pl.BlockSpec((1, TB, FEAT), lambda n, t: (n, t, 0)),
            pl.BlockSpec((FEAT, C), lambda n, t: (0, 0)),
            pl.BlockSpec((1, C), lambda n, t: (0, 0)),
            pl.BlockSpec((1, UP, V), lambda n, t: (n, 0, 0)),
            pl.BlockSpec((1, UP, V), lambda n, t: (n, 0, 0)),
            pl.BlockSpec((V, C), lambda n, t: (0, 0)),
            pl.BlockSpec((C, V), lambda n, t: (0, 0)),
            pl.BlockSpec((1, V), lambda n, t: (0, 0)),
        ],
        out_specs=[
            pl.BlockSpec((1, TB, UP), lambda n, t: (n, t, 0)),
            pl.BlockSpec((1, TB, UP), lambda n, t: (n, t, 0)),
        ],
        out_shape=[
            jax.ShapeDtypeStruct((N, T, UP), f32),
            jax.ShapeDtypeStruct((N, T, UP), f32),
        ],
        scratch_shapes=[pltpu.VMEM((UP, C), jnp.float32)],
    )(x.astype(jnp.bfloat16), W_enc.astype(jnp.bfloat16),
      b_enc.reshape(1, C).astype(f32), oh, ohs, emb.astype(f32),
      W_join.astype(jnp.bfloat16), b_join.reshape(1, V).astype(f32))

    sb = _shear(jnp.transpose(lpb, (0, 2, 1)), 1, T)
    lpsT = jnp.transpose(lps, (0, 2, 1))
    ls2 = jnp.pad(lpsT, ((0, 0), (1, 0), (0, 0)),
                  constant_values=NEG)[:, :UP]
    ss = _shear(ls2, 0, T)

    packed = jnp.concatenate([sb, ss], axis=1)

    alphas = pl.pallas_call(
        _phase2,
        out_shape=jax.ShapeDtypeStruct((D_TOT, N, UP), f32),
    )(packed)

    n_idx = jnp.arange(N)
    dn = x_lens - 1 + y_lens
    final_alpha = alphas[dn, n_idx, y_lens]
    final_blank = lpb[n_idx, x_lens - 1, y_lens]
    return -jnp.sum(final_alpha + final_blank)


# speedup vs baseline: 1.2826x; 1.2560x over previous
import jax
import jax.numpy as jnp
from jax.experimental import pallas as pl
from jax.experimental.pallas import tpu as pltpu

N, T, FEAT, C, U, V = 4, 512, 80, 256, 48, 256
BLANK = 0
UP = 56
TB = 128
NEG = -1e30
D_TOT = T + U + 1


def _phase1(x_ref, we_ref, be_ref, oh_ref, ohs_ref, emb_ref, wj_ref, bj_ref,
            lpb_ref, lps_ref, dec_ref):
    xb = x_ref[0]
    enc = jnp.dot(xb, we_ref[...], preferred_element_type=jnp.float32) + be_ref[0]

    @pl.when(pl.program_id(1) == 0)
    def _():
        oh = oh_ref[0]
        dec_ref[...] = jnp.dot(oh, emb_ref[...],
                               preferred_element_type=jnp.float32)

    dec = dec_ref[...]
    joint = jnp.tanh(enc.astype(jnp.bfloat16)[:, None, :]
                     + dec.astype(jnp.bfloat16)[None, :, :])
    logits = jnp.dot(joint.reshape(TB * UP, C), wj_ref[...],
                     preferred_element_type=jnp.float32) + bj_ref[0]
    l3 = logits.reshape(TB, UP, V)
    lse = jnp.log(jnp.sum(jnp.exp(l3), axis=2))
    iota_v = jax.lax.broadcasted_iota(jnp.int32, (TB, UP, V), 2)
    lpb = jnp.sum(jnp.where(iota_v == BLANK, l3, 0.0), axis=2) - lse
    sym = jnp.sum(l3 * ohs_ref[0][None], axis=2) - lse
    ucol = jax.lax.broadcasted_iota(jnp.int32, (TB, UP), 1)
    lpb_ref[0] = jnp.where(ucol <= U, lpb, NEG)
    lps_ref[0] = jnp.where(ucol < U, sym, NEG)


def _phase2(pk_ref, out_ref, wk_ref):
    u_iota = jax.lax.broadcasted_iota(jnp.int32, (N, UP), 1)
    a0 = jnp.where(u_iota == 0, 0.0, NEG)
    out_ref[0] = a0

    allr = pk_ref[...]
    sbo, sbe = allr[:, 0:N], allr[:, N:2 * N]
    sso, sse = allr[:, 2 * N:3 * N], allr[:, 3 * N:4 * N]

    def shu(v):
        return jnp.concatenate(
            [jnp.full(v.shape[:-1] + (1,), NEG, jnp.float32), v[..., :-1]],
            axis=-1)

    w0 = sbo + sbe
    w1 = jnp.logaddexp(sso + sbe, shu(sbo) + sse)
    w2 = shu(sso) + sse
    wk_ref[...] = jnp.concatenate([w0, w1, w2], axis=1)

    def body(j, a):
        wk = wk_ref[j]
        pk = pk_ref[j]
        v0, v1, v2 = wk[0:N], wk[N:2 * N], wk[2 * N:3 * N]
        sb1, ss1 = pk[0:N], pk[2 * N:3 * N]
        a1 = jnp.concatenate(
            [jnp.full((N, 1), NEG, jnp.float32), a[:, :-1]], axis=1)
        a2 = jnp.concatenate(
            [jnp.full((N, 2), NEG, jnp.float32), a[:, :-2]], axis=1)
        p1 = a + sb1
        q1 = a1 + ss1
        mo = jnp.maximum(p1, q1)
        a_odd = mo + jnp.log1p(jnp.exp(jnp.minimum(p1, q1) - mo))
        out_ref[2 * j + 1] = a_odd
        t0 = a + v0
        t1 = a1 + v1
        t2 = a2 + v2
        m = jnp.maximum(jnp.maximum(t0, t1), t2)
        s = jnp.exp(t0 - m) + jnp.exp(t1 - m) + jnp.exp(t2 - m)
        a_new = m + jnp.log(s)
        out_ref[2 * j + 2] = a_new
        return a_new

    jax.lax.fori_loop(0, (D_TOT - 1) // 2, body, a0, unroll=4)


def _shear(arrT, left_pad, width):
    w = width + left_pad
    p = jnp.pad(arrT, ((0, 0), (0, 0), (left_pad, (w + UP + 1) - w)),
                constant_values=NEG)
    flat = p.reshape(N, UP * (w + UP + 1))[:, :UP * (w + UP)]
    sh = flat.reshape(N, UP, w + UP)[:, :, :D_TOT]
    return jnp.transpose(sh, (2, 0, 1))


def kernel(x, x_lens, y_padded, y_lens, W_enc, b_enc, emb, W_join, b_join):
    f32 = jnp.float32
    sos_y = jnp.concatenate(
        [jnp.zeros((N, 1), y_padded.dtype), y_padded], axis=1)
    sos_pad = jnp.pad(sos_y, ((0, 0), (0, UP - (U + 1))))
    vio = jnp.arange(V, dtype=sos_pad.dtype)
    oh = (sos_pad[:, :, None] == vio).astype(f32)
    yp_pad = jnp.pad(y_padded, ((0, 0), (0, UP - U)), constant_values=-1)
    ohs = (yp_pad[:, :, None] == vio).astype(f32)

    lpb, lps = pl.pallas_call(
        _phase1,
        grid=(N, T // TB),
        in_specs=[
            pl.BlockSpec((1, TB, FEAT), lambda n, t: (n, t, 0)),
            pl.BlockSpec((FEAT, C), lambda n, t: (0, 0)),
            pl.BlockSpec((1, C), lambda n, t: (0, 0)),
            pl.BlockSpec((1, UP, V), lambda n, t: (n, 0, 0)),
            pl.BlockSpec((1, UP, V), lambda n, t: (n, 0, 0)),
            pl.BlockSpec((V, C), lambda n, t: (0, 0)),
            pl.BlockSpec((C, V), lambda n, t: (0, 0)),
            pl.BlockSpec((1, V), lambda n, t: (0, 0)),
        ],
        out_specs=[
            pl.BlockSpec((1, TB, UP), lambda n, t: (n, t, 0)),
            pl.BlockSpec((1, TB, UP), lambda n, t: (n, t, 0)),
        ],
        out_shape=[
            jax.ShapeDtypeStruct((N, T, UP), f32),
            jax.ShapeDtypeStruct((N, T, UP), f32),
        ],
        scratch_shapes=[pltpu.VMEM((UP, C), jnp.float32)],
    )(x.astype(jnp.bfloat16), W_enc.astype(jnp.bfloat16),
      b_enc.reshape(1, C).astype(f32), oh, ohs, emb.astype(f32),
      W_join.astype(jnp.bfloat16), b_join.reshape(1, V).astype(f32))

    sb = _shear(jnp.transpose(lpb, (0, 2, 1)), 1, T)
    lpsT = jnp.transpose(lps, (0, 2, 1))
    ls2 = jnp.pad(lpsT, ((0, 0), (1, 0), (0, 0)),
                  constant_values=NEG)[:, :UP]
    ss = _shear(ls2, 0, T)

    dh = (D_TOT - 1) // 2
    packed = jnp.concatenate(
        [sb[1:].reshape(dh, 2 * N, UP), ss[1:].reshape(dh, 2 * N, UP)],
        axis=1)

    alphas = pl.pallas_call(
        _phase2,
        out_shape=jax.ShapeDtypeStruct((D_TOT, N, UP), f32),
        scratch_shapes=[pltpu.VMEM((dh, 3 * N, UP), jnp.float32)],
    )(packed)

    n_idx = jnp.arange(N)
    dn = x_lens - 1 + y_lens
    final_alpha = alphas[dn, n_idx, y_lens]
    final_blank = lpb[n_idx, x_lens - 1, y_lens]
    return -jnp.sum(final_alpha + final_blank)
